# baseline (speedup 1.0000x reference)
import functools

import numpy as np

import jax
import jax.numpy as jnp
from jax import lax
from jax.experimental import pallas as pl
from jax.experimental.pallas import tpu as pltpu
from jax.experimental.pallas import tpu_sc as plsc

B = 16384
EMB = 32
GRP = 8
GW = GRP * EMB
PWI = GW // 2
NC, NS = 2, 16
NW = NC * NS
BPW = B // NW
CHUNK = 128
NCHUNK = BPW // CHUNK
BR = 1024


def _num_blocks(n):
    return (n // GRP + BR - 1) // BR


def _trans_body(*refs):
    ins = refs[:2 * GRP]
    oa_ref, ob_ref = refs[2 * GRP:]
    eye = jnp.eye(EMB, dtype=jnp.float32)

    def mxu_t(x):
        return lax.dot_general(x[...], eye, (((0,), (0,)), ((), ())),
                               preferred_element_type=jnp.float32)

    def tr(x, j):
        return mxu_t(x) if j % 2 else x[...].T

    ci = lax.broadcasted_iota(jnp.int32, (GW, PWI), 0)
    ki = lax.broadcasted_iota(jnp.int32, (GW, PWI), 1)
    sel_e = (ci == 2 * ki).astype(jnp.float32)
    sel_o = (ci == 2 * ki + 1).astype(jnp.float32)

    def pack(y):
        ev = jnp.dot(y, sel_e, preferred_element_type=jnp.float32)
        od = jnp.dot(y, sel_o, preferred_element_type=jnp.float32)
        ue = lax.shift_right_logical(
            lax.bitcast_convert_type(ev, jnp.uint32), jnp.uint32(16))
        uo = lax.bitwise_and(
            lax.bitcast_convert_type(od, jnp.uint32),
            jnp.uint32(0xFFFF0000))
        return lax.bitcast_convert_type(ue | uo, jnp.int32)

    oa_ref[...] = pack(jnp.concatenate(
        [tr(x, j) for j, x in enumerate(ins[:GRP])], axis=1))
    ob_ref[...] = pack(jnp.concatenate(
        [tr(x, j) for j, x in enumerate(ins[GRP:])], axis=1))


def _trans_pair(tTa, tTb):
    n = tTa.shape[1]
    nb = _num_blocks(n)
    last = (n - 1) // BR

    def spec(j):
        return pl.BlockSpec(
            (EMB, BR), lambda i, j=j: (0, jnp.minimum(j * nb + i, last)))

    out_sdt = jax.ShapeDtypeStruct((nb * BR, PWI), jnp.int32)
    return pl.pallas_call(
        _trans_body,
        grid=(nb,),
        in_specs=[spec(j) for j in range(GRP)] * 2,
        out_specs=[pl.BlockSpec((BR, PWI), lambda i: (i, 0))] * 2,
        out_shape=[out_sdt, out_sdt],
    )(*([tTa] * GRP + [tTb] * GRP))


def _sc_gather(user_gidx, movie_gidx, gu_t, gm_t, mu_t, mm_t):
    mesh = plsc.VectorSubcoreMesh(core_axis_name="c", subcore_axis_name="s")
    out_type = tuple(jax.ShapeDtypeStruct((B, PWI), jnp.int32)
                     for _ in range(4))
    scratch = [
        pltpu.VMEM((NCHUNK, CHUNK), jnp.int32),
        pltpu.VMEM((NCHUNK, CHUNK), jnp.int32),
        pltpu.VMEM((BPW, PWI), jnp.int32),
        pltpu.SemaphoreType.DMA,
    ]

    @functools.partial(pl.kernel, mesh=mesh, out_type=out_type,
                       scratch_types=scratch)
    def k(uids_hbm, mids_hbm, t0, t1, t2, t3, o0, o1, o2, o3,
          idx_u, idx_m, buf, sem):
        wid = lax.axis_index("s") * NC + lax.axis_index("c")
        irow = wid * NCHUNK
        base = wid * BPW
        pltpu.sync_copy(uids_hbm.at[pl.ds(irow, NCHUNK)], idx_u)
        pltpu.sync_copy(mids_hbm.at[pl.ds(irow, NCHUNK)], idx_m)
        for tbl, idx, out in ((t0, idx_u, o0), (t1, idx_m, o1),
                              (t2, idx_u, o2), (t3, idx_m, o3)):
            copies = [pltpu.async_copy(
                tbl.at[idx.at[j]], buf.at[pl.ds(j * CHUNK, CHUNK)], sem)
                for j in range(NCHUNK)]
            for cp in copies:
                cp.wait()
            pltpu.sync_copy(buf, out.at[pl.ds(base, BPW)])

    return k(user_gidx, movie_gidx, gu_t, gm_t, mu_t, mm_t)


BLK = 2048


HG = EMB // 2


def _pick(rows, sel):
    w = rows[:, 0:HG]
    for j in range(1, GRP):
        w = jnp.where(sel == j, rows[:, j * HG:(j + 1) * HG], w)
    even = lax.bitcast_convert_type(w << 16, jnp.float32)
    odd = lax.bitcast_convert_type(w & jnp.int32(-65536), jnp.float32)
    return jnp.concatenate([even, odd], axis=1)


def _tc_body(gu, gm, mu, mm, su, sm, w1, b1, w2, b2, w3, b3, wo, bo, out):
    sel_u = su[...]
    sel_m = sm[...]
    gmf = _pick(gu[...], sel_u) * _pick(gm[...], sel_m)
    x = jnp.concatenate([_pick(mu[...], sel_u), _pick(mm[...], sel_m)],
                        axis=1)
    h = jnp.maximum(jnp.dot(x, w1[...], preferred_element_type=jnp.float32)
                    + b1[...], 0.0)
    h = jnp.maximum(jnp.dot(h, w2[...], preferred_element_type=jnp.float32)
                    + b2[...], 0.0)
    h = jnp.maximum(jnp.dot(h, w3[...], preferred_element_type=jnp.float32)
                    + b3[...], 0.0)
    comb = jnp.concatenate([gmf, h], axis=1)
    z = jnp.dot(comb, wo[...], preferred_element_type=jnp.float32) + bo[...]
    out[...] = jax.nn.sigmoid(z)


def _tc_dense(gu, gm, mu, mm, su, sm,
              w1t, b1, w2t, b2, w3t, b3, wot, bo):
    row_spec = pl.BlockSpec((BLK, PWI), lambda i: (i, 0))
    sel_spec = pl.BlockSpec((BLK, 1), lambda i: (i, 0))

    def whole(shape):
        return pl.BlockSpec(shape, lambda i: tuple(0 for _ in shape))

    return pl.pallas_call(
        _tc_body,
        grid=(B // BLK,),
        in_specs=[row_spec, row_spec, row_spec, row_spec,
                  sel_spec, sel_spec,
                  whole((64, 64)), whole((1, 64)),
                  whole((64, 32)), whole((1, 32)),
                  whole((32, 16)), whole((1, 16)),
                  whole((48, 1)), whole((1, 1))],
        out_specs=pl.BlockSpec((BLK, 1), lambda i: (i, 0)),
        out_shape=jax.ShapeDtypeStruct((B, 1), jnp.float32),
    )(gu, gm, mu, mm, su, sm, w1t, b1, w2t, b2, w3t, b3, wot, bo)


def _pack_outside(tbl):
    m = tbl.reshape(-1, GW)
    ev = m[:, 0::2]
    od = m[:, 1::2]
    ue = lax.shift_right_logical(
        lax.bitcast_convert_type(ev, jnp.uint32), jnp.uint32(16))
    uo = lax.bitwise_and(lax.bitcast_convert_type(od, jnp.uint32),
                         jnp.uint32(0xFFFF0000))
    return lax.bitcast_convert_type(ue | uo, jnp.int32)


_PERM = np.concatenate([np.arange(0, EMB, 2), np.arange(1, EMB, 2)])


def kernel(user_ids, movie_ids, gmf_user_emb, gmf_movie_emb,
           mlp_user_emb, mlp_movie_emb, W1, b1, W2, b2, W3, b3, Wo, bo):
    qu = _num_blocks(gmf_user_emb.shape[0]) * BR
    ug = (user_ids % qu).reshape(B // CHUNK, CHUNK)
    su = (user_ids // qu).reshape(B, 1)
    mg = (movie_ids // GRP).reshape(B // CHUNK, CHUNK)
    sm = (movie_ids % GRP).reshape(B, 1)
    gu_t, mu_t = _trans_pair(gmf_user_emb.T, mlp_user_emb.T)
    gu, gm, mu, mm = _sc_gather(
        ug, mg, gu_t, _pack_outside(gmf_movie_emb),
        mu_t, _pack_outside(mlp_movie_emb))
    inperm = np.concatenate([_PERM, EMB + _PERM])
    woperm = np.concatenate([_PERM, EMB + np.arange(16)])
    out = _tc_dense(gu, gm, mu, mm, su, sm,
                    W1.T[inperm], b1.reshape(1, 64),
                    W2.T, b2.reshape(1, 32),
                    W3.T, b3.reshape(1, 16),
                    Wo.T[woperm], bo.reshape(1, 1))
    return out.reshape(B)


# speedup vs baseline: 2.2522x; 2.2522x over previous
import functools

import jax
import jax.numpy as jnp
from jax import lax
from jax.experimental import pallas as pl
from jax.experimental.pallas import tpu as pltpu
from jax.experimental.pallas import tpu_sc as plsc

B = 16384
EMB = 32
GRP = 4
GW = GRP * EMB
NC, NS = 2, 16
NW = NC * NS
BPW = B // NW
CHUNK = 128
NCHUNK = BPW // CHUNK
BR = 2048


def _num_blocks(n):
    return (n // GRP + BR - 1) // BR


def _trans_body(a0, a1, a2, a3, b0, b1, b2, b3, oa_ref, ob_ref):
    eye = jnp.eye(EMB, dtype=jnp.float32)

    def mxu_t(x):
        return lax.dot_general(x[...], eye, (((0,), (0,)), ((), ())),
                               preferred_element_type=jnp.float32)

    oa_ref[...] = jnp.concatenate(
        [a0[...].T, mxu_t(a1), a2[...].T, mxu_t(a3)], axis=1)
    ob_ref[...] = jnp.concatenate(
        [b0[...].T, mxu_t(b1), b2[...].T, mxu_t(b3)], axis=1)


def _trans_pair(tTa, tTb):
    n = tTa.shape[1]
    nb = _num_blocks(n)
    last = (n - 1) // BR

    def spec(j):
        return pl.BlockSpec(
            (EMB, BR), lambda i, j=j: (0, jnp.minimum(j * nb + i, last)))

    out_sdt = jax.ShapeDtypeStruct((nb * BR, GW), jnp.float32)
    return pl.pallas_call(
        _trans_body,
        grid=(nb,),
        in_specs=[spec(0), spec(1), spec(2), spec(3)] * 2,
        out_specs=[pl.BlockSpec((BR, GW), lambda i: (i, 0))] * 2,
        out_shape=[out_sdt, out_sdt],
    )(tTa, tTa, tTa, tTa, tTb, tTb, tTb, tTb)


def _sc_gather(user_gidx, movie_gidx, gu_t, gm_t, mu_t, mm_t):
    mesh = plsc.VectorSubcoreMesh(core_axis_name="c", subcore_axis_name="s")
    out_type = tuple(jax.ShapeDtypeStruct((B, GW), jnp.float32)
                     for _ in range(4))
    scratch = [
        pltpu.VMEM((NCHUNK, CHUNK), jnp.int32),
        pltpu.VMEM((NCHUNK, CHUNK), jnp.int32),
        pltpu.VMEM((BPW, GW), jnp.float32),
        pltpu.SemaphoreType.DMA,
    ]

    @functools.partial(pl.kernel, mesh=mesh, out_type=out_type,
                       scratch_types=scratch)
    def k(uids_hbm, mids_hbm, t0, t1, t2, t3, o0, o1, o2, o3,
          idx_u, idx_m, buf, sem):
        wid = lax.axis_index("s") * NC + lax.axis_index("c")
        irow = wid * NCHUNK
        base = wid * BPW
        pltpu.sync_copy(uids_hbm.at[pl.ds(irow, NCHUNK)], idx_u)
        pltpu.sync_copy(mids_hbm.at[pl.ds(irow, NCHUNK)], idx_m)
        for tbl, idx, out in ((t0, idx_u, o0), (t1, idx_m, o1),
                              (t2, idx_u, o2), (t3, idx_m, o3)):
            copies = [pltpu.async_copy(
                tbl.at[idx.at[j]], buf.at[pl.ds(j * CHUNK, CHUNK)], sem)
                for j in range(NCHUNK)]
            for cp in copies:
                cp.wait()
            pltpu.sync_copy(buf, out.at[pl.ds(base, BPW)])

    return k(user_gidx, movie_gidx, gu_t, gm_t, mu_t, mm_t)


BLK = 4096


def _pick(rows, sel):
    out = jnp.where(sel == 0, rows[:, 0:EMB], rows[:, EMB:2 * EMB])
    out = jnp.where(sel == 2, rows[:, 2 * EMB:3 * EMB], out)
    return jnp.where(sel == 3, rows[:, 3 * EMB:4 * EMB], out)


def _tc_body(gu, gm, mu, mm, su, sm, w1, b1, w2, b2, w3, b3, wo, bo, out):
    sel_u = su[...]
    sel_m = sm[...]
    gmf = _pick(gu[...], sel_u) * _pick(gm[...], sel_m)
    x = jnp.concatenate([_pick(mu[...], sel_u), _pick(mm[...], sel_m)],
                        axis=1)
    h = jnp.maximum(jnp.dot(x, w1[...], preferred_element_type=jnp.float32)
                    + b1[...], 0.0)
    h = jnp.maximum(jnp.dot(h, w2[...], preferred_element_type=jnp.float32)
                    + b2[...], 0.0)
    h = jnp.maximum(jnp.dot(h, w3[...], preferred_element_type=jnp.float32)
                    + b3[...], 0.0)
    comb = jnp.concatenate([gmf, h], axis=1)
    z = jnp.dot(comb, wo[...], preferred_element_type=jnp.float32) + bo[...]
    out[...] = jax.nn.sigmoid(z)


def _tc_dense(gu, gm, mu, mm, su, sm,
              w1t, b1, w2t, b2, w3t, b3, wot, bo):
    row_spec = pl.BlockSpec((BLK, GW), lambda i: (i, 0))
    sel_spec = pl.BlockSpec((BLK, 1), lambda i: (i, 0))

    def whole(shape):
        return pl.BlockSpec(shape, lambda i: tuple(0 for _ in shape))

    return pl.pallas_call(
        _tc_body,
        grid=(B // BLK,),
        in_specs=[row_spec, row_spec, row_spec, row_spec,
                  sel_spec, sel_spec,
                  whole((64, 64)), whole((1, 64)),
                  whole((64, 32)), whole((1, 32)),
                  whole((32, 16)), whole((1, 16)),
                  whole((48, 1)), whole((1, 1))],
        out_specs=pl.BlockSpec((BLK, 1), lambda i: (i, 0)),
        out_shape=jax.ShapeDtypeStruct((B, 1), jnp.float32),
    )(gu, gm, mu, mm, su, sm, w1t, b1, w2t, b2, w3t, b3, wot, bo)


def kernel(user_ids, movie_ids, gmf_user_emb, gmf_movie_emb,
           mlp_user_emb, mlp_movie_emb, W1, b1, W2, b2, W3, b3, Wo, bo):
    qu = _num_blocks(gmf_user_emb.shape[0]) * BR
    ug = (user_ids % qu).reshape(B // CHUNK, CHUNK)
    su = (user_ids // qu).reshape(B, 1)
    mg = (movie_ids // GRP).reshape(B // CHUNK, CHUNK)
    sm = (movie_ids % GRP).reshape(B, 1)
    gu_t, mu_t = _trans_pair(gmf_user_emb.T, mlp_user_emb.T)
    gu, gm, mu, mm = _sc_gather(
        ug, mg, gu_t, gmf_movie_emb.reshape(-1, GW),
        mu_t, mlp_movie_emb.reshape(-1, GW))
    out = _tc_dense(gu, gm, mu, mm, su, sm,
                    W1.T, b1.reshape(1, 64),
                    W2.T, b2.reshape(1, 32),
                    W3.T, b3.reshape(1, 16),
                    Wo.T, bo.reshape(1, 1))
    return out.reshape(B)


# speedup vs baseline: 2.2866x; 1.0153x over previous
import functools

import jax
import jax.numpy as jnp
from jax import lax
from jax.experimental import pallas as pl
from jax.experimental.pallas import tpu as pltpu
from jax.experimental.pallas import tpu_sc as plsc

B = 16384
EMB = 32
GRP = 4
GW = GRP * EMB
NC, NS = 2, 16
NW = NC * NS
BPW = B // NW
CHUNK = 128
NCHUNK = BPW // CHUNK
BR = 4096


def _num_blocks(n):
    return (n // GRP + BR - 1) // BR


def _trans_body(a0, a1, a2, a3, b0, b1, b2, b3, oa_ref, ob_ref):
    eye = jnp.eye(EMB, dtype=jnp.float32)

    def mxu_t(x):
        return lax.dot_general(x[...], eye, (((0,), (0,)), ((), ())),
                               preferred_element_type=jnp.float32)

    oa_ref[...] = jnp.concatenate(
        [a0[...].T, mxu_t(a1), a2[...].T, mxu_t(a3)], axis=1)
    ob_ref[...] = jnp.concatenate(
        [b0[...].T, mxu_t(b1), b2[...].T, mxu_t(b3)], axis=1)


def _trans_pair(tTa, tTb):
    n = tTa.shape[1]
    nb = _num_blocks(n)
    last = (n - 1) // BR

    def spec(j):
        return pl.BlockSpec(
            (EMB, BR), lambda i, j=j: (0, jnp.minimum(j * nb + i, last)))

    out_sdt = jax.ShapeDtypeStruct((nb * BR, GW), jnp.float32)
    return pl.pallas_call(
        _trans_body,
        grid=(nb,),
        in_specs=[spec(0), spec(1), spec(2), spec(3)] * 2,
        out_specs=[pl.BlockSpec((BR, GW), lambda i: (i, 0))] * 2,
        out_shape=[out_sdt, out_sdt],
    )(tTa, tTa, tTa, tTa, tTb, tTb, tTb, tTb)


def _sc_gather(user_gidx, movie_gidx, gu_t, gm_t, mu_t, mm_t):
    mesh = plsc.VectorSubcoreMesh(core_axis_name="c", subcore_axis_name="s")
    out_type = tuple(jax.ShapeDtypeStruct((B, GW), jnp.float32)
                     for _ in range(4))
    scratch = [
        pltpu.VMEM((NCHUNK, CHUNK), jnp.int32),
        pltpu.VMEM((NCHUNK, CHUNK), jnp.int32),
        pltpu.VMEM((BPW, GW), jnp.float32),
        pltpu.SemaphoreType.DMA,
    ]

    @functools.partial(pl.kernel, mesh=mesh, out_type=out_type,
                       scratch_types=scratch)
    def k(uids_hbm, mids_hbm, t0, t1, t2, t3, o0, o1, o2, o3,
          idx_u, idx_m, buf, sem):
        wid = lax.axis_index("s") * NC + lax.axis_index("c")
        irow = wid * NCHUNK
        base = wid * BPW
        pltpu.sync_copy(uids_hbm.at[pl.ds(irow, NCHUNK)], idx_u)
        pltpu.sync_copy(mids_hbm.at[pl.ds(irow, NCHUNK)], idx_m)
        for tbl, idx, out in ((t0, idx_u, o0), (t1, idx_m, o1),
                              (t2, idx_u, o2), (t3, idx_m, o3)):
            copies = [pltpu.async_copy(
                tbl.at[idx.at[j]], buf.at[pl.ds(j * CHUNK, CHUNK)], sem)
                for j in range(NCHUNK)]
            for cp in copies:
                cp.wait()
            pltpu.sync_copy(buf, out.at[pl.ds(base, BPW)])

    return k(user_gidx, movie_gidx, gu_t, gm_t, mu_t, mm_t)


BLK = 4096


def _pick(rows, sel):
    out = jnp.where(sel == 0, rows[:, 0:EMB], rows[:, EMB:2 * EMB])
    out = jnp.where(sel == 2, rows[:, 2 * EMB:3 * EMB], out)
    return jnp.where(sel == 3, rows[:, 3 * EMB:4 * EMB], out)


def _tc_body(gu, gm, mu, mm, su, sm, w1, b1, w2, b2, w3, b3, wo, bo, out):
    sel_u = su[...]
    sel_m = sm[...]
    gmf = _pick(gu[...], sel_u) * _pick(gm[...], sel_m)
    x = jnp.concatenate([_pick(mu[...], sel_u), _pick(mm[...], sel_m)],
                        axis=1)
    h = jnp.maximum(jnp.dot(x, w1[...], preferred_element_type=jnp.float32)
                    + b1[...], 0.0)
    h = jnp.maximum(jnp.dot(h, w2[...], preferred_element_type=jnp.float32)
                    + b2[...], 0.0)
    h = jnp.maximum(jnp.dot(h, w3[...], preferred_element_type=jnp.float32)
                    + b3[...], 0.0)
    comb = jnp.concatenate([gmf, h], axis=1)
    z = jnp.dot(comb, wo[...], preferred_element_type=jnp.float32) + bo[...]
    out[...] = jax.nn.sigmoid(z)


def _tc_dense(gu, gm, mu, mm, su, sm,
              w1t, b1, w2t, b2, w3t, b3, wot, bo):
    row_spec = pl.BlockSpec((BLK, GW), lambda i: (i, 0))
    sel_spec = pl.BlockSpec((BLK, 1), lambda i: (i, 0))

    def whole(shape):
        return pl.BlockSpec(shape, lambda i: tuple(0 for _ in shape))

    return pl.pallas_call(
        _tc_body,
        grid=(B // BLK,),
        in_specs=[row_spec, row_spec, row_spec, row_spec,
                  sel_spec, sel_spec,
                  whole((64, 64)), whole((1, 64)),
                  whole((64, 32)), whole((1, 32)),
                  whole((32, 16)), whole((1, 16)),
                  whole((48, 1)), whole((1, 1))],
        out_specs=pl.BlockSpec((BLK, 1), lambda i: (i, 0)),
        out_shape=jax.ShapeDtypeStruct((B, 1), jnp.float32),
    )(gu, gm, mu, mm, su, sm, w1t, b1, w2t, b2, w3t, b3, wot, bo)


def kernel(user_ids, movie_ids, gmf_user_emb, gmf_movie_emb,
           mlp_user_emb, mlp_movie_emb, W1, b1, W2, b2, W3, b3, Wo, bo):
    qu = _num_blocks(gmf_user_emb.shape[0]) * BR
    ug = (user_ids % qu).reshape(B // CHUNK, CHUNK)
    su = (user_ids // qu).reshape(B, 1)
    mg = (movie_ids // GRP).reshape(B // CHUNK, CHUNK)
    sm = (movie_ids % GRP).reshape(B, 1)
    gu_t, mu_t = _trans_pair(gmf_user_emb.T, mlp_user_emb.T)
    gu, gm, mu, mm = _sc_gather(
        ug, mg, gu_t, gmf_movie_emb.reshape(-1, GW),
        mu_t, mlp_movie_emb.reshape(-1, GW))
    out = _tc_dense(gu, gm, mu, mm, su, sm,
                    W1.T, b1.reshape(1, 64),
                    W2.T, b2.reshape(1, 32),
                    W3.T, b3.reshape(1, 16),
                    Wo.T, bo.reshape(1, 1))
    return out.reshape(B)
